# baseline (speedup 1.0000x reference)
import jax
import jax.numpy as jnp
from jax.experimental import pallas as pl

N_EXPERTS = 8
TOP_K = 2
BLOCK_T = 8192


def _gate_kernel(x_ref, w_ref, probs_ref, tv_ref, ti_ref):
    x = x_ref[...]
    w = w_ref[...]
    scores = jax.lax.dot_general(
        w, x, (((1,), (1,)), ((), ())), preferred_element_type=jnp.float32
    )
    m = jnp.max(scores, axis=0, keepdims=True)
    e = jnp.exp(scores - m)
    s = jnp.sum(e, axis=0, keepdims=True)
    probs = e / s
    probs_ref[...] = probs

    v1 = jnp.max(probs, axis=0, keepdims=True)
    i1 = jnp.argmax(probs, axis=0).reshape(1, -1)
    row = jax.lax.broadcasted_iota(jnp.int32, probs.shape, 0)
    masked = jnp.where(row == i1, -jnp.inf, probs)
    v2 = jnp.max(masked, axis=0, keepdims=True)
    i2 = jnp.argmax(masked, axis=0).reshape(1, -1)
    denom = v1 + v2 + 1e-9
    tv_ref[...] = jnp.concatenate([v1 / denom, v2 / denom], axis=0)
    ti_ref[...] = jnp.concatenate([i1, i2], axis=0).astype(jnp.int32)


def kernel(x, weight):
    n_tok, dim = x.shape
    n_exp = weight.shape[0]
    grid = (n_tok // BLOCK_T,)
    probs_t, tv_t, ti_t = pl.pallas_call(
        _gate_kernel,
        grid=grid,
        in_specs=[
            pl.BlockSpec((BLOCK_T, dim), lambda i: (i, 0)),
            pl.BlockSpec((n_exp, dim), lambda i: (0, 0)),
        ],
        out_specs=[
            pl.BlockSpec((n_exp, BLOCK_T), lambda i: (0, i)),
            pl.BlockSpec((TOP_K, BLOCK_T), lambda i: (0, i)),
            pl.BlockSpec((TOP_K, BLOCK_T), lambda i: (0, i)),
        ],
        out_shape=[
            jax.ShapeDtypeStruct((n_exp, n_tok), jnp.float32),
            jax.ShapeDtypeStruct((TOP_K, n_tok), jnp.float32),
            jax.ShapeDtypeStruct((TOP_K, n_tok), jnp.int32),
        ],
    )(x, weight)
    return tv_t.T, ti_t.T, probs_t.T


# speedup vs baseline: 1.0727x; 1.0727x over previous
import jax
import jax.numpy as jnp
from jax.experimental import pallas as pl

N_EXPERTS = 8
TOP_K = 2
BLOCK_T = 4096


def _gate_kernel(x_ref, w_ref, probs_ref, tv_ref, ti_ref):
    x = x_ref[...]
    w = w_ref[...]
    scores = jax.lax.dot_general(
        w, x, (((1,), (1,)), ((), ())), preferred_element_type=jnp.float32
    )
    m = jnp.max(scores, axis=0, keepdims=True)
    e = jnp.exp(scores - m)
    s = jnp.sum(e, axis=0, keepdims=True)
    probs = e / s
    probs_ref[...] = probs

    v1 = jnp.max(probs, axis=0, keepdims=True)
    i1 = jnp.argmax(probs, axis=0).reshape(1, -1)
    row = jax.lax.broadcasted_iota(jnp.int32, probs.shape, 0)
    masked = jnp.where(row == i1, -jnp.inf, probs)
    v2 = jnp.max(masked, axis=0, keepdims=True)
    i2 = jnp.argmax(masked, axis=0).reshape(1, -1)
    denom = v1 + v2 + 1e-9
    tv_ref[...] = jnp.concatenate([v1 / denom, v2 / denom], axis=0)
    ti_ref[...] = jnp.concatenate([i1, i2], axis=0).astype(jnp.int32)


def kernel(x, weight):
    n_tok, dim = x.shape
    n_exp = weight.shape[0]
    grid = (n_tok // BLOCK_T,)
    probs_t, tv_t, ti_t = pl.pallas_call(
        _gate_kernel,
        grid=grid,
        in_specs=[
            pl.BlockSpec((BLOCK_T, dim), lambda i: (i, 0)),
            pl.BlockSpec((n_exp, dim), lambda i: (0, 0)),
        ],
        out_specs=[
            pl.BlockSpec((n_exp, BLOCK_T), lambda i: (0, i)),
            pl.BlockSpec((TOP_K, BLOCK_T), lambda i: (0, i)),
            pl.BlockSpec((TOP_K, BLOCK_T), lambda i: (0, i)),
        ],
        out_shape=[
            jax.ShapeDtypeStruct((n_exp, n_tok), jnp.float32),
            jax.ShapeDtypeStruct((TOP_K, n_tok), jnp.float32),
            jax.ShapeDtypeStruct((TOP_K, n_tok), jnp.int32),
        ],
    )(x, weight)
    return tv_t.T, ti_t.T, probs_t.T
